# baseline (speedup 1.0000x reference)
import functools

import jax
import jax.numpy as jnp
from jax import lax
from jax.experimental import pallas as pl
from jax.experimental.pallas import tpu as pltpu
from jax.experimental.pallas import tpu_sc as plsc

R = 512
F = 16
L = 16
NC = 2
NS = 16
NW = NC * NS
CH = 256
HC = 128
BN = 4096
TBL = 3 * R * R


def _minmax_body(x_ref, mn_ref, mx_ref):
    i = pl.program_id(0)
    xb = x_ref[...]
    mn = jnp.broadcast_to(jnp.min(xb, axis=1, keepdims=True), (3, 128))
    mx = jnp.broadcast_to(jnp.max(xb, axis=1, keepdims=True), (3, 128))

    @pl.when(i == 0)
    def _():
        mn_ref[...] = mn
        mx_ref[...] = mx

    @pl.when(i > 0)
    def _():
        mn_ref[...] = jnp.minimum(mn_ref[...], mn)
        mx_ref[...] = jnp.maximum(mx_ref[...], mx)


def _idxw_body(x_ref, mn_ref, mx_ref, idx_ref, wx_ref, wy_ref):
    xb = x_ref[...]
    mn3 = mn_ref[:, 0:1]
    mx3 = mx_ref[:, 0:1]
    scale = (R - 1.0) / (mx3 - mn3 + 1e-8)
    ic = (xb - mn3) * scale
    i0f = jnp.clip(jnp.floor(ic), 0.0, R - 2.0)
    w = ic - i0f
    ii = i0f.astype(jnp.int32)
    idx_ref[0:1, :] = ii[1:2, :] * R + ii[0:1, :]
    idx_ref[1:2, :] = R * R + ii[2:3, :] * R + ii[0:1, :]
    idx_ref[2:3, :] = 2 * R * R + ii[2:3, :] * R + ii[1:2, :]
    wx_ref[0:1, :] = w[0:1, :]
    wx_ref[1:2, :] = w[0:1, :]
    wx_ref[2:3, :] = w[1:2, :]
    wy_ref[0:1, :] = w[1:2, :]
    wy_ref[1:2, :] = w[2:3, :]
    wy_ref[2:3, :] = w[2:3, :]


CC = 1024
CPT = R * R // NW
NT_STEPS = 3 * (CPT // CC)


def _make_transpose_kernel():
    mesh = plsc.VectorSubcoreMesh(core_axis_name="c", subcore_axis_name="s")

    @functools.partial(
        pl.kernel,
        mesh=mesh,
        compiler_params=pltpu.CompilerParams(use_tc_tiling_on_sc=False,
                                             needs_layout_passes=False),
        out_type=jax.ShapeDtypeStruct((TBL, 2 * F), jnp.float32),
        scratch_types=(
            [pltpu.VMEM((F, CC), jnp.float32) for _ in range(4)]
            + [pltpu.VMEM((CC, 2 * F), jnp.float32) for _ in range(2)]
            + [pltpu.SemaphoreType.DMA for _ in range(4)]
        ),
    )
    def tr_kernel(p0, p1, p2, outh, ia, ib, ja, jb, oa, ob,
                  si_a, si_b, so_a, so_b):
        wid = lax.axis_index("s") * NC + lax.axis_index("c")
        planes = (p0, p1, p2)
        ibuf = (ia, ib)
        jbuf = (ja, jb)
        obuf = (oa, ob)
        sin = (si_a, si_b)
        sout = (so_a, so_b)
        lane = lax.iota(jnp.int32, 16)

        def bases(s):
            p, t = divmod(s, CPT // CC)
            cbase = wid * CPT + t * CC
            bstart = jnp.minimum(cbase + R, R * R - CC)
            return p, cbase, bstart

        def fire_in(s):
            p, cbase, bstart = bases(s)
            pltpu.async_copy(planes[p].at[:, pl.ds(cbase, CC)],
                             ibuf[s % 2], sin[s % 2])
            pltpu.async_copy(planes[p].at[:, pl.ds(bstart, CC)],
                             jbuf[s % 2], sin[s % 2])

        def wait_in(s):
            par = s % 2
            pltpu.make_async_copy(planes[0].at[:, pl.ds(0, CC)],
                                  ibuf[par], sin[par]).wait()
            pltpu.make_async_copy(planes[0].at[:, pl.ds(0, CC)],
                                  jbuf[par], sin[par]).wait()

        def step(s):
            par = s % 2
            wait_in(s)
            if s >= 2:
                pltpu.make_async_copy(outh.at[pl.ds(0, CC), :], obuf[par],
                                      sout[par]).wait()
            if s + 1 < NT_STEPS:
                fire_in(s + 1)
            p, cbase, bstart = bases(s)
            d = cbase + R - bstart
            ib_ = ibuf[par]
            jb_ = jbuf[par]
            ob_ = obuf[par]

            def tgrp(t2, cin):
                for u in range(16):
                    c = t2 * 16 + u
                    row = plsc.load_gather(ib_, [lane, jnp.full((16,), c,
                                                                jnp.int32)])
                    cj = jnp.minimum(c + d, CC - 1)
                    row2 = plsc.load_gather(jb_, [lane, jnp.full((16,), cj,
                                                                 jnp.int32)])
                    ob_[c, pl.ds(0, F)] = row
                    ob_[c, pl.ds(F, F)] = row2
                return cin

            lax.fori_loop(0, CC // 16, tgrp, 0)
            base = p * R * R + wid * CPT + (s % (CPT // CC)) * CC
            pltpu.async_copy(ob_, outh.at[pl.ds(base, CC), :], sout[par])

        fire_in(0)
        for s in range(NT_STEPS):
            step(s)
        pltpu.make_async_copy(outh.at[pl.ds(0, CC), :], obuf[0],
                              sout[0]).wait()
        pltpu.make_async_copy(outh.at[pl.ds(0, CC), :], obuf[1],
                              sout[1]).wait()

    return tr_kernel


def _make_sc_kernel(n_pad):
    nw_pts = n_pad // NW
    n_chunks = nw_pts // CH
    n_steps = 3 * n_chunks
    loop_iters = (n_steps - 1) // 6
    mesh = plsc.VectorSubcoreMesh(core_axis_name="c", subcore_axis_name="s")

    @functools.partial(
        pl.kernel,
        mesh=mesh,
        compiler_params=pltpu.CompilerParams(use_tc_tiling_on_sc=False),
        out_type=jax.ShapeDtypeStruct((n_pad, 3 * F), jnp.float32),
        scratch_types=(
            [pltpu.VMEM((CH,), jnp.int32) for _ in range(3)]
            + [pltpu.VMEM((CH,), jnp.float32) for _ in range(6)]
            + [pltpu.VMEM((CH, 2 * F), jnp.float32) for _ in range(4)]
            + [pltpu.VMEM((CH, 3 * F), jnp.float32) for _ in range(2)]
            + [pltpu.SemaphoreType.DMA for _ in range(5)]
        ),
    )
    def sc_kernel(table, idxh, wxh, wyh, outh,
                  i0a, i0b, i0c, wxa, wxb, wxc, wya, wyb, wyc,
                  ca0, ca1, cb0, cb1,
                  oca, ocb, sin_a, sin_b, sin_c, sg_a, sg_b):
        wid = lax.axis_index("s") * NC + lax.axis_index("c")
        i0 = (i0a, i0b, i0c)
        wxv = (wxa, wxb, wxc)
        wyv = (wya, wyb, wyc)
        corners = ((ca0, ca1), (cb0, cb1))
        oc = (oca, ocb)
        sem_in = (sin_a, sin_b, sin_c)
        sem_g = (sg_a, sg_b)
        tbls = (table, table.at[pl.ds(1, TBL - 1)])

        def fire_in(g, p):
            base = p * n_pad + wid * nw_pts + g * CH
            pltpu.async_copy(idxh.at[pl.ds(base, CH)], i0[p], sem_in[p])
            pltpu.async_copy(wxh.at[pl.ds(base, CH)], wxv[p], sem_in[p])
            pltpu.async_copy(wyh.at[pl.ds(base, CH)], wyv[p], sem_in[p])

        def wait_in(p):
            pltpu.make_async_copy(idxh.at[pl.ds(0, CH)], i0[p],
                                  sem_in[p]).wait()
            pltpu.make_async_copy(wxh.at[pl.ds(0, CH)], wxv[p],
                                  sem_in[p]).wait()
            pltpu.make_async_copy(wyh.at[pl.ds(0, CH)], wyv[p],
                                  sem_in[p]).wait()

        def fire_gathers(p, par):
            for c in range(2):
                for h in range(2):
                    iv = i0[p].at[pl.ds(h * HC, HC)]
                    dv = corners[par][c].at[pl.ds(h * HC, HC), :]
                    pltpu.async_copy(tbls[c].at[iv], dv, sem_g[par])

        def wait_gathers(par):
            for c in range(2):
                pltpu.make_async_copy(table.at[pl.ds(0, CH)],
                                      corners[par][c], sem_g[par]).wait()

        def combine(p, par, poc):
            cp0, cp1 = corners[par]
            ocr = oc[poc]
            wxr = wxv[p]
            wyr = wyv[p]

            def grp(t, cin):
                s16 = pl.ds(t * L, L)
                wxg = wxr[s16]
                wyg = wyr[s16]
                ax = 1.0 - wxg
                ay = 1.0 - wyg
                a0 = ax * ay
                a1 = wxg * ay
                a2 = ax * wyg
                a3 = wxg * wyg
                for u in range(L):
                    pt = t * L + u
                    acc = (cp0[pt, pl.ds(0, F)] * a0[u]
                           + cp1[pt, pl.ds(0, F)] * a1[u]
                           + cp0[pt, pl.ds(F, F)] * a2[u]
                           + cp1[pt, pl.ds(F, F)] * a3[u])
                    ocr[pt, pl.ds(p * F, F)] = acc
                return cin

            lax.fori_loop(0, CH // L, grp, 0)

        def store(g, poc):
            base = wid * nw_pts + g * CH
            pltpu.sync_copy(oc[poc], outh.at[pl.ds(base, CH), :])

        fire_in(0, 0)
        fire_in(0, 1)
        fire_in(0, 2)
        wait_in(0)
        fire_gathers(0, 0)

        def body(k2, carry):
            for j in range(6):
                wait_in((1 + j) % 3)
                fire_gathers((1 + j) % 3, (1 + j) % 2)
                wait_gathers(j % 2)
                combine(j % 3, j % 2, (j // 3) % 2)
                fire_in(2 * k2 + 1 + j // 3, j % 3)
                if j % 3 == 2:
                    store(2 * k2 + j // 3, (j // 3) % 2)
            return carry

        lax.fori_loop(0, loop_iters, body, 0)

        for s in range(6 * loop_iters + 1, n_steps):
            wait_in(s % 3)
            fire_gathers(s % 3, s % 2)
            sp = s - 1
            wait_gathers(sp % 2)
            combine(sp % 3, sp % 2, (sp // 3) % 2)
            if s + 2 < n_steps:
                fire_in((s + 2) // 3, (s + 2) % 3)
            if sp % 3 == 2:
                store(sp // 3, (sp // 3) % 2)
        sp = n_steps - 1
        wait_gathers(sp % 2)
        combine(sp % 3, sp % 2, (sp // 3) % 2)
        store(sp // 3, (sp // 3) % 2)

    return sc_kernel


def kernel(x, plane_xy, plane_xz, plane_yz):
    n = x.shape[0]
    blk = NW * CH * 2
    n_pad = ((n + blk - 1) // blk) * blk
    n_pad = ((n_pad + BN - 1) // BN) * BN

    table = _make_transpose_kernel()(plane_xy[0].reshape(F, R * R),
                                     plane_xz[0].reshape(F, R * R),
                                     plane_yz[0].reshape(F, R * R))

    x_t = jnp.pad(jnp.transpose(x), ((0, 0), (0, n_pad - n)), mode="edge")

    grid = n_pad // BN
    mn, mx = pl.pallas_call(
        _minmax_body,
        grid=(grid,),
        in_specs=[pl.BlockSpec((3, BN), lambda i: (0, i))],
        out_specs=[pl.BlockSpec((3, 128), lambda i: (0, 0)),
                   pl.BlockSpec((3, 128), lambda i: (0, 0))],
        out_shape=[jax.ShapeDtypeStruct((3, 128), jnp.float32),
                   jax.ShapeDtypeStruct((3, 128), jnp.float32)],
    )(x_t)

    idx, wx, wy = pl.pallas_call(
        _idxw_body,
        grid=(grid,),
        in_specs=[pl.BlockSpec((3, BN), lambda i: (0, i)),
                  pl.BlockSpec((3, 128), lambda i: (0, 0)),
                  pl.BlockSpec((3, 128), lambda i: (0, 0))],
        out_specs=[pl.BlockSpec((3, BN), lambda i: (0, i)),
                   pl.BlockSpec((3, BN), lambda i: (0, i)),
                   pl.BlockSpec((3, BN), lambda i: (0, i))],
        out_shape=[jax.ShapeDtypeStruct((3, n_pad), jnp.int32),
                   jax.ShapeDtypeStruct((3, n_pad), jnp.float32),
                   jax.ShapeDtypeStruct((3, n_pad), jnp.float32)],
    )(x_t, mn, mx)

    out = _make_sc_kernel(n_pad)(table, idx.reshape(-1), wx.reshape(-1),
                                 wy.reshape(-1))
    return out[:n]


# speedup vs baseline: 1.0870x; 1.0870x over previous
import functools

import jax
import jax.numpy as jnp
from jax import lax
from jax.experimental import pallas as pl
from jax.experimental.pallas import tpu as pltpu
from jax.experimental.pallas import tpu_sc as plsc

R = 512
F = 16
L = 16
NC = 2
NS = 16
NW = NC * NS
CH = 256
HC = 128
BN = 4096
TBL = 3 * R * R


def _minmax_body(x_ref, mn_ref, mx_ref):
    i = pl.program_id(0)
    xb = x_ref[...]
    mn = jnp.broadcast_to(jnp.min(xb, axis=1, keepdims=True), (3, 128))
    mx = jnp.broadcast_to(jnp.max(xb, axis=1, keepdims=True), (3, 128))

    @pl.when(i == 0)
    def _():
        mn_ref[...] = mn
        mx_ref[...] = mx

    @pl.when(i > 0)
    def _():
        mn_ref[...] = jnp.minimum(mn_ref[...], mn)
        mx_ref[...] = jnp.maximum(mx_ref[...], mx)


def _idxw_body(x_ref, mn_ref, mx_ref, idx_ref, wx_ref, wy_ref):
    xb = x_ref[...]
    mn3 = mn_ref[:, 0:1]
    mx3 = mx_ref[:, 0:1]
    scale = (R - 1.0) / (mx3 - mn3 + 1e-8)
    ic = (xb - mn3) * scale
    i0f = jnp.clip(jnp.floor(ic), 0.0, R - 2.0)
    w = ic - i0f
    ii = i0f.astype(jnp.int32)
    idx_ref[0:1, :] = ii[1:2, :] * R + ii[0:1, :]
    idx_ref[1:2, :] = R * R + ii[2:3, :] * R + ii[0:1, :]
    idx_ref[2:3, :] = 2 * R * R + ii[2:3, :] * R + ii[1:2, :]
    wx_ref[0:1, :] = w[0:1, :]
    wx_ref[1:2, :] = w[0:1, :]
    wx_ref[2:3, :] = w[1:2, :]
    wy_ref[0:1, :] = w[1:2, :]
    wy_ref[1:2, :] = w[2:3, :]
    wy_ref[2:3, :] = w[2:3, :]


CC = 1024
CPT = R * R // NW
NT_STEPS = 3 * (CPT // CC)


def _make_transpose_kernel():
    mesh = plsc.VectorSubcoreMesh(core_axis_name="c", subcore_axis_name="s")

    @functools.partial(
        pl.kernel,
        mesh=mesh,
        compiler_params=pltpu.CompilerParams(use_tc_tiling_on_sc=False,
                                             needs_layout_passes=False),
        out_type=jax.ShapeDtypeStruct((TBL, 2 * F), jnp.float32),
        scratch_types=(
            [pltpu.VMEM((F, CC), jnp.float32) for _ in range(4)]
            + [pltpu.VMEM((CC, 2 * F), jnp.float32) for _ in range(2)]
            + [pltpu.SemaphoreType.DMA for _ in range(4)]
        ),
    )
    def tr_kernel(p0, p1, p2, outh, ia, ib, ja, jb, oa, ob,
                  si_a, si_b, so_a, so_b):
        wid = lax.axis_index("s") * NC + lax.axis_index("c")
        planes = (p0, p1, p2)
        ibuf = (ia, ib)
        jbuf = (ja, jb)
        obuf = (oa, ob)
        sin = (si_a, si_b)
        sout = (so_a, so_b)
        lane = lax.iota(jnp.int32, 16)

        def bases(s):
            p, t = divmod(s, CPT // CC)
            cbase = wid * CPT + t * CC
            bstart = jnp.minimum(cbase + R, R * R - CC)
            return p, cbase, bstart

        def fire_in(s):
            p, cbase, bstart = bases(s)
            pltpu.async_copy(planes[p].at[:, pl.ds(cbase, CC)],
                             ibuf[s % 2], sin[s % 2])
            pltpu.async_copy(planes[p].at[:, pl.ds(bstart, CC)],
                             jbuf[s % 2], sin[s % 2])

        def wait_in(s):
            par = s % 2
            pltpu.make_async_copy(planes[0].at[:, pl.ds(0, CC)],
                                  ibuf[par], sin[par]).wait()
            pltpu.make_async_copy(planes[0].at[:, pl.ds(0, CC)],
                                  jbuf[par], sin[par]).wait()

        def step(s):
            par = s % 2
            wait_in(s)
            if s >= 2:
                pltpu.make_async_copy(outh.at[pl.ds(0, CC), :], obuf[par],
                                      sout[par]).wait()
            if s + 1 < NT_STEPS:
                fire_in(s + 1)
            p, cbase, bstart = bases(s)
            d = cbase + R - bstart
            ib_ = ibuf[par]
            jb_ = jbuf[par]
            ob_ = obuf[par]

            def tgrp(t2, cin):
                for u in range(16):
                    c = t2 * 16 + u
                    row = plsc.load_gather(ib_, [lane, jnp.full((16,), c,
                                                                jnp.int32)])
                    cj = jnp.minimum(c + d, CC - 1)
                    row2 = plsc.load_gather(jb_, [lane, jnp.full((16,), cj,
                                                                 jnp.int32)])
                    ob_[c, pl.ds(0, F)] = row
                    ob_[c, pl.ds(F, F)] = row2
                return cin

            lax.fori_loop(0, CC // 16, tgrp, 0)
            base = p * R * R + wid * CPT + (s % (CPT // CC)) * CC
            pltpu.async_copy(ob_, outh.at[pl.ds(base, CC), :], sout[par])

        fire_in(0)
        for s in range(NT_STEPS):
            step(s)
        pltpu.make_async_copy(outh.at[pl.ds(0, CC), :], obuf[0],
                              sout[0]).wait()
        pltpu.make_async_copy(outh.at[pl.ds(0, CC), :], obuf[1],
                              sout[1]).wait()

    return tr_kernel


def _make_sc_kernel(n_pad):
    nw_pts = n_pad // NW
    n_chunks = nw_pts // CH
    n_steps = 3 * n_chunks
    loop_iters = (n_steps - 1) // 6
    mesh = plsc.VectorSubcoreMesh(core_axis_name="c", subcore_axis_name="s")

    @functools.partial(
        pl.kernel,
        mesh=mesh,
        compiler_params=pltpu.CompilerParams(use_tc_tiling_on_sc=False),
        out_type=jax.ShapeDtypeStruct((n_pad, 3 * F), jnp.float32),
        scratch_types=(
            [pltpu.VMEM((CH,), jnp.int32) for _ in range(3)]
            + [pltpu.VMEM((CH,), jnp.float32) for _ in range(6)]
            + [pltpu.VMEM((CH, 2 * F), jnp.float32) for _ in range(4)]
            + [pltpu.VMEM((CH, 3 * F), jnp.float32) for _ in range(2)]
            + [pltpu.SemaphoreType.DMA for _ in range(5)]
        ),
    )
    def sc_kernel(table, idxh, wxh, wyh, outh,
                  i0a, i0b, i0c, wxa, wxb, wxc, wya, wyb, wyc,
                  ca0, ca1, cb0, cb1,
                  oca, ocb, sin_a, sin_b, sin_c, sg_a, sg_b):
        wid = lax.axis_index("s") * NC + lax.axis_index("c")
        i0 = (i0a, i0b, i0c)
        wxv = (wxa, wxb, wxc)
        wyv = (wya, wyb, wyc)
        corners = ((ca0, ca1), (cb0, cb1))
        oc = (oca, ocb)
        sem_in = (sin_a, sin_b, sin_c)
        sem_g = (sg_a, sg_b)
        tbls = (table, table.at[pl.ds(1, TBL - 1)])

        def fire_in(g, p):
            base = p * n_pad + wid * nw_pts + g * CH
            pltpu.async_copy(idxh.at[pl.ds(base, CH)], i0[p], sem_in[p])
            pltpu.async_copy(wxh.at[pl.ds(base, CH)], wxv[p], sem_in[p])
            pltpu.async_copy(wyh.at[pl.ds(base, CH)], wyv[p], sem_in[p])

        def wait_in(p):
            pltpu.make_async_copy(idxh.at[pl.ds(0, CH)], i0[p],
                                  sem_in[p]).wait()
            pltpu.make_async_copy(wxh.at[pl.ds(0, CH)], wxv[p],
                                  sem_in[p]).wait()
            pltpu.make_async_copy(wyh.at[pl.ds(0, CH)], wyv[p],
                                  sem_in[p]).wait()

        def fire_gathers(p, par):
            for c in range(2):
                for h in range(2):
                    iv = i0[p].at[pl.ds(h * HC, HC)]
                    dv = corners[par][c].at[pl.ds(h * HC, HC), :]
                    pltpu.async_copy(tbls[c].at[iv], dv, sem_g[par])

        def wait_gathers(par):
            for c in range(2):
                pltpu.make_async_copy(table.at[pl.ds(0, CH)],
                                      corners[par][c], sem_g[par]).wait()

        def combine(p, par, poc):
            cp0, cp1 = corners[par]
            ocr = oc[poc]
            wxr = wxv[p]
            wyr = wyv[p]

            def grp(t, cin):
                s16 = pl.ds(t * L, L)
                wxg = wxr[s16]
                wyg = wyr[s16]
                ax = 1.0 - wxg
                ay = 1.0 - wyg
                a0 = ax * ay
                a1 = wxg * ay
                a2 = ax * wyg
                a3 = wxg * wyg
                for u in range(0, L, 4):
                    pts = [t * L + u + k for k in range(4)]
                    r0 = [cp0[pt, pl.ds(0, F)] for pt in pts]
                    r1 = [cp1[pt, pl.ds(0, F)] for pt in pts]
                    r2 = [cp0[pt, pl.ds(F, F)] for pt in pts]
                    r3 = [cp1[pt, pl.ds(F, F)] for pt in pts]
                    m0 = [r0[k] * a0[u + k] for k in range(4)]
                    m1 = [r1[k] * a1[u + k] for k in range(4)]
                    m2 = [r2[k] * a2[u + k] for k in range(4)]
                    m3 = [r3[k] * a3[u + k] for k in range(4)]
                    for k in range(4):
                        ocr[pts[k], pl.ds(p * F, F)] = ((m0[k] + m1[k])
                                                        + (m2[k] + m3[k]))
                return cin

            lax.fori_loop(0, CH // L, grp, 0)

        def store(g, poc):
            base = wid * nw_pts + g * CH
            pltpu.sync_copy(oc[poc], outh.at[pl.ds(base, CH), :])

        fire_in(0, 0)
        fire_in(0, 1)
        fire_in(0, 2)
        wait_in(0)
        fire_gathers(0, 0)

        def body(k2, carry):
            for j in range(6):
                wait_in((1 + j) % 3)
                fire_gathers((1 + j) % 3, (1 + j) % 2)
                wait_gathers(j % 2)
                combine(j % 3, j % 2, (j // 3) % 2)
                fire_in(2 * k2 + 1 + j // 3, j % 3)
                if j % 3 == 2:
                    store(2 * k2 + j // 3, (j // 3) % 2)
            return carry

        lax.fori_loop(0, loop_iters, body, 0)

        for s in range(6 * loop_iters + 1, n_steps):
            wait_in(s % 3)
            fire_gathers(s % 3, s % 2)
            sp = s - 1
            wait_gathers(sp % 2)
            combine(sp % 3, sp % 2, (sp // 3) % 2)
            if s + 2 < n_steps:
                fire_in((s + 2) // 3, (s + 2) % 3)
            if sp % 3 == 2:
                store(sp // 3, (sp // 3) % 2)
        sp = n_steps - 1
        wait_gathers(sp % 2)
        combine(sp % 3, sp % 2, (sp // 3) % 2)
        store(sp // 3, (sp // 3) % 2)

    return sc_kernel


def kernel(x, plane_xy, plane_xz, plane_yz):
    n = x.shape[0]
    blk = NW * CH * 2
    n_pad = ((n + blk - 1) // blk) * blk
    n_pad = ((n_pad + BN - 1) // BN) * BN

    table = _make_transpose_kernel()(plane_xy[0].reshape(F, R * R),
                                     plane_xz[0].reshape(F, R * R),
                                     plane_yz[0].reshape(F, R * R))

    x_t = jnp.pad(jnp.transpose(x), ((0, 0), (0, n_pad - n)), mode="edge")

    grid = n_pad // BN
    mn, mx = pl.pallas_call(
        _minmax_body,
        grid=(grid,),
        in_specs=[pl.BlockSpec((3, BN), lambda i: (0, i))],
        out_specs=[pl.BlockSpec((3, 128), lambda i: (0, 0)),
                   pl.BlockSpec((3, 128), lambda i: (0, 0))],
        out_shape=[jax.ShapeDtypeStruct((3, 128), jnp.float32),
                   jax.ShapeDtypeStruct((3, 128), jnp.float32)],
    )(x_t)

    idx, wx, wy = pl.pallas_call(
        _idxw_body,
        grid=(grid,),
        in_specs=[pl.BlockSpec((3, BN), lambda i: (0, i)),
                  pl.BlockSpec((3, 128), lambda i: (0, 0)),
                  pl.BlockSpec((3, 128), lambda i: (0, 0))],
        out_specs=[pl.BlockSpec((3, BN), lambda i: (0, i)),
                   pl.BlockSpec((3, BN), lambda i: (0, i)),
                   pl.BlockSpec((3, BN), lambda i: (0, i))],
        out_shape=[jax.ShapeDtypeStruct((3, n_pad), jnp.int32),
                   jax.ShapeDtypeStruct((3, n_pad), jnp.float32),
                   jax.ShapeDtypeStruct((3, n_pad), jnp.float32)],
    )(x_t, mn, mx)

    out = _make_sc_kernel(n_pad)(table, idx.reshape(-1), wx.reshape(-1),
                                 wy.reshape(-1))
    return out[:n]


# speedup vs baseline: 1.7529x; 1.6126x over previous
import functools

import jax
import jax.numpy as jnp
from jax import lax
from jax.experimental import pallas as pl
from jax.experimental.pallas import tpu as pltpu
from jax.experimental.pallas import tpu_sc as plsc

R = 512
F = 16
L = 16
NC = 2
NS = 16
NW = NC * NS
CH = 256
HC = 128
BN = 4096
TBL = 3 * R * R


def _minmax_body(x_ref, mn_ref, mx_ref):
    i = pl.program_id(0)
    xb = x_ref[...]
    mn = jnp.broadcast_to(jnp.min(xb, axis=1, keepdims=True), (3, 128))
    mx = jnp.broadcast_to(jnp.max(xb, axis=1, keepdims=True), (3, 128))

    @pl.when(i == 0)
    def _():
        mn_ref[...] = mn
        mx_ref[...] = mx

    @pl.when(i > 0)
    def _():
        mn_ref[...] = jnp.minimum(mn_ref[...], mn)
        mx_ref[...] = jnp.maximum(mx_ref[...], mx)


def _idxw_body(x_ref, mn_ref, mx_ref, idx_ref, wx_ref, wy_ref):
    xb = x_ref[...]
    mn3 = mn_ref[:, 0:1]
    mx3 = mx_ref[:, 0:1]
    scale = (R - 1.0) / (mx3 - mn3 + 1e-8)
    ic = (xb - mn3) * scale
    i0f = jnp.clip(jnp.floor(ic), 0.0, R - 2.0)
    w = ic - i0f
    ii = i0f.astype(jnp.int32)
    idx_ref[0:1, :] = ii[1:2, :] * R + ii[0:1, :]
    idx_ref[1:2, :] = R * R + ii[2:3, :] * R + ii[0:1, :]
    idx_ref[2:3, :] = 2 * R * R + ii[2:3, :] * R + ii[1:2, :]
    wx_ref[0:1, :] = w[0:1, :]
    wx_ref[1:2, :] = w[0:1, :]
    wx_ref[2:3, :] = w[1:2, :]
    wy_ref[0:1, :] = w[1:2, :]
    wy_ref[1:2, :] = w[2:3, :]
    wy_ref[2:3, :] = w[2:3, :]


CC = 1024
CPT = R * R // NW
NT_STEPS = 3 * (CPT // CC)


def _make_transpose_kernel():
    mesh = plsc.VectorSubcoreMesh(core_axis_name="c", subcore_axis_name="s")

    @functools.partial(
        pl.kernel,
        mesh=mesh,
        compiler_params=pltpu.CompilerParams(use_tc_tiling_on_sc=False,
                                             needs_layout_passes=False),
        out_type=jax.ShapeDtypeStruct((TBL, 4 * F), jnp.bfloat16),
        scratch_types=(
            [pltpu.VMEM((F, CC), jnp.float32) for _ in range(4)]
            + [pltpu.VMEM((CC, 4 * F), jnp.bfloat16) for _ in range(2)]
            + [pltpu.SemaphoreType.DMA for _ in range(4)]
        ),
    )
    def tr_kernel(p0, p1, p2, outh, ia, ib, ja, jb, oa, ob,
                  si_a, si_b, so_a, so_b):
        wid = lax.axis_index("s") * NC + lax.axis_index("c")
        planes = (p0, p1, p2)
        ibuf = (ia, ib)
        jbuf = (ja, jb)
        obuf = (oa, ob)
        sin = (si_a, si_b)
        sout = (so_a, so_b)
        lane = lax.iota(jnp.int32, 16)

        def bases(s):
            p, t = divmod(s, CPT // CC)
            cbase = wid * CPT + t * CC
            bstart = jnp.minimum(cbase + R, R * R - CC)
            return p, cbase, bstart

        def fire_in(s):
            p, cbase, bstart = bases(s)
            pltpu.async_copy(planes[p].at[:, pl.ds(cbase, CC)],
                             ibuf[s % 2], sin[s % 2])
            pltpu.async_copy(planes[p].at[:, pl.ds(bstart, CC)],
                             jbuf[s % 2], sin[s % 2])

        def wait_in(s):
            par = s % 2
            pltpu.make_async_copy(planes[0].at[:, pl.ds(0, CC)],
                                  ibuf[par], sin[par]).wait()
            pltpu.make_async_copy(planes[0].at[:, pl.ds(0, CC)],
                                  jbuf[par], sin[par]).wait()

        def step(s):
            par = s % 2
            wait_in(s)
            if s >= 2:
                pltpu.make_async_copy(outh.at[pl.ds(0, CC), :], obuf[par],
                                      sout[par]).wait()
            if s + 1 < NT_STEPS:
                fire_in(s + 1)
            p, cbase, bstart = bases(s)
            d = cbase + R - bstart
            ib_ = ibuf[par]
            jb_ = jbuf[par]
            ob_ = obuf[par]

            @plsc.parallel_loop(0, CC // 8)
            def tgrp(t2):
                cs = [t2 * 8 + k for k in range(9)]
                i_idx = [jnp.full((16,), jnp.minimum(c, CC - 1), jnp.int32)
                         for c in cs]
                j_idx = [jnp.full((16,), jnp.minimum(c + d, CC - 1),
                                  jnp.int32) for c in cs]
                ra = [plsc.load_gather(ib_, [lane, ix]) for ix in i_idx]
                rb = [plsc.load_gather(jb_, [lane, ix]) for ix in j_idx]
                pk = [plsc.pack(ra[k], rb[k],
                                format=plsc.PackFormat.INTERLEAVED)
                      for k in range(9)]
                for k in range(8):
                    ob_[cs[k], pl.ds(0, 2 * F)] = pk[k]
                    ob_[cs[k], pl.ds(2 * F, 2 * F)] = pk[k + 1]
            base = p * R * R + wid * CPT + (s % (CPT // CC)) * CC
            pltpu.async_copy(ob_, outh.at[pl.ds(base, CC), :], sout[par])

        fire_in(0)
        for s in range(NT_STEPS):
            step(s)
        pltpu.make_async_copy(outh.at[pl.ds(0, CC), :], obuf[0],
                              sout[0]).wait()
        pltpu.make_async_copy(outh.at[pl.ds(0, CC), :], obuf[1],
                              sout[1]).wait()

    return tr_kernel


def _make_sc_kernel(n, n_pad):
    nw_pts = n_pad // NW
    n_chunks = nw_pts // CH
    n_steps = 3 * n_chunks
    loop_iters = (n_steps - 1) // 6
    pn = n - CH
    mesh = plsc.VectorSubcoreMesh(core_axis_name="c", subcore_axis_name="s")

    @functools.partial(
        pl.kernel,
        mesh=mesh,
        compiler_params=pltpu.CompilerParams(use_tc_tiling_on_sc=False,
                                             needs_layout_passes=False),
        out_type=jax.ShapeDtypeStruct((n, 3 * F), jnp.float32),
        scratch_types=(
            [pltpu.VMEM((CH,), jnp.int32) for _ in range(3)]
            + [pltpu.VMEM((CH,), jnp.float32) for _ in range(6)]
            + [pltpu.VMEM((CH, 4 * F), jnp.bfloat16) for _ in range(2)]
            + [pltpu.VMEM((CH, 3 * F), jnp.float32) for _ in range(2)]
            + [pltpu.SemaphoreType.DMA for _ in range(5)]
        ),
    )
    def sc_kernel(table, idxh, wxh, wyh, outh,
                  i0a, i0b, i0c, wxa, wxb, wxc, wya, wyb, wyc,
                  ca0, cb0,
                  oca, ocb, sin_a, sin_b, sin_c, sg_a, sg_b):
        wid = lax.axis_index("s") * NC + lax.axis_index("c")
        i0 = (i0a, i0b, i0c)
        wxv = (wxa, wxb, wxc)
        wyv = (wya, wyb, wyc)
        corners = (ca0, cb0)
        oc = (oca, ocb)
        sem_in = (sin_a, sin_b, sin_c)
        sem_g = (sg_a, sg_b)

        def pbase(g):
            return jnp.minimum(wid * nw_pts + g * CH, pn)

        def fire_in(g, p):
            base = p * n_pad + pbase(g)
            pltpu.async_copy(idxh.at[pl.ds(base, CH)], i0[p], sem_in[p])
            pltpu.async_copy(wxh.at[pl.ds(base, CH)], wxv[p], sem_in[p])
            pltpu.async_copy(wyh.at[pl.ds(base, CH)], wyv[p], sem_in[p])

        def wait_in(p):
            pltpu.make_async_copy(idxh.at[pl.ds(0, CH)], i0[p],
                                  sem_in[p]).wait()
            pltpu.make_async_copy(wxh.at[pl.ds(0, CH)], wxv[p],
                                  sem_in[p]).wait()
            pltpu.make_async_copy(wyh.at[pl.ds(0, CH)], wyv[p],
                                  sem_in[p]).wait()

        def fire_gathers(p, par):
            for h in range(2):
                iv = i0[p].at[pl.ds(h * HC, HC)]
                dv = corners[par].at[pl.ds(h * HC, HC), :]
                pltpu.async_copy(table.at[iv], dv, sem_g[par])

        def wait_gathers(par):
            pltpu.make_async_copy(table.at[pl.ds(0, CH)],
                                  corners[par], sem_g[par]).wait()

        def combine(p, par, poc):
            cp = corners[par]
            ocr = oc[poc]
            wxr = wxv[p]
            wyr = wyv[p]

            @plsc.parallel_loop(0, CH // L)
            def grp(t):
                s16 = pl.ds(t * L, L)
                wxg = wxr[s16]
                wyg = wyr[s16]
                ax = 1.0 - wxg
                ay = 1.0 - wyg
                a0 = ax * ay
                a1 = wxg * ay
                a2 = ax * wyg
                a3 = wxg * wyg
                for u in range(0, L, 4):
                    pts = [t * L + u + k for k in range(4)]
                    u0 = [plsc.unpack(cp[pt, pl.ds(0, 2 * F)],
                                      format=plsc.PackFormat.INTERLEAVED)
                          for pt in pts]
                    u1 = [plsc.unpack(cp[pt, pl.ds(2 * F, 2 * F)],
                                      format=plsc.PackFormat.INTERLEAVED)
                          for pt in pts]
                    m0 = [u0[k][0] * a0[u + k] for k in range(4)]
                    m1 = [u1[k][0] * a1[u + k] for k in range(4)]
                    m2 = [u0[k][1] * a2[u + k] for k in range(4)]
                    m3 = [u1[k][1] * a3[u + k] for k in range(4)]
                    for k in range(4):
                        ocr[pts[k], pl.ds(p * F, F)] = ((m0[k] + m1[k])
                                                        + (m2[k] + m3[k]))

        def store(g, poc):
            pltpu.sync_copy(oc[poc], outh.at[pl.ds(pbase(g), CH), :])

        fire_in(0, 0)
        fire_in(0, 1)
        fire_in(0, 2)
        wait_in(0)
        fire_gathers(0, 0)

        def body(k2, carry):
            for j in range(6):
                wait_in((1 + j) % 3)
                fire_gathers((1 + j) % 3, (1 + j) % 2)
                wait_gathers(j % 2)
                combine(j % 3, j % 2, (j // 3) % 2)
                fire_in(2 * k2 + 1 + j // 3, j % 3)
                if j % 3 == 2:
                    store(2 * k2 + j // 3, (j // 3) % 2)
            return carry

        lax.fori_loop(0, loop_iters, body, 0)

        for s in range(6 * loop_iters + 1, n_steps):
            wait_in(s % 3)
            fire_gathers(s % 3, s % 2)
            sp = s - 1
            wait_gathers(sp % 2)
            combine(sp % 3, sp % 2, (sp // 3) % 2)
            if s + 2 < n_steps:
                fire_in((s + 2) // 3, (s + 2) % 3)
            if sp % 3 == 2:
                store(sp // 3, (sp // 3) % 2)
        sp = n_steps - 1
        wait_gathers(sp % 2)
        combine(sp % 3, sp % 2, (sp // 3) % 2)
        store(sp // 3, (sp // 3) % 2)

    return sc_kernel


def kernel(x, plane_xy, plane_xz, plane_yz):
    n = x.shape[0]
    blk = NW * CH * 2
    n_pad = ((n + blk - 1) // blk) * blk
    n_pad = ((n_pad + BN - 1) // BN) * BN

    table = _make_transpose_kernel()(plane_xy[0].reshape(F, R * R),
                                     plane_xz[0].reshape(F, R * R),
                                     plane_yz[0].reshape(F, R * R))

    x_t = jnp.pad(jnp.transpose(x), ((0, 0), (0, n_pad - n)), mode="edge")

    grid = n_pad // BN
    mn, mx = pl.pallas_call(
        _minmax_body,
        grid=(grid,),
        in_specs=[pl.BlockSpec((3, BN), lambda i: (0, i))],
        out_specs=[pl.BlockSpec((3, 128), lambda i: (0, 0)),
                   pl.BlockSpec((3, 128), lambda i: (0, 0))],
        out_shape=[jax.ShapeDtypeStruct((3, 128), jnp.float32),
                   jax.ShapeDtypeStruct((3, 128), jnp.float32)],
    )(x_t)

    idx, wx, wy = pl.pallas_call(
        _idxw_body,
        grid=(grid,),
        in_specs=[pl.BlockSpec((3, BN), lambda i: (0, i)),
                  pl.BlockSpec((3, 128), lambda i: (0, 0)),
                  pl.BlockSpec((3, 128), lambda i: (0, 0))],
        out_specs=[pl.BlockSpec((3, BN), lambda i: (0, i)),
                   pl.BlockSpec((3, BN), lambda i: (0, i)),
                   pl.BlockSpec((3, BN), lambda i: (0, i))],
        out_shape=[jax.ShapeDtypeStruct((3, n_pad), jnp.int32),
                   jax.ShapeDtypeStruct((3, n_pad), jnp.float32),
                   jax.ShapeDtypeStruct((3, n_pad), jnp.float32)],
    )(x_t, mn, mx)

    return _make_sc_kernel(n, n_pad)(table, idx.reshape(-1), wx.reshape(-1),
                                     wy.reshape(-1))
